# baseline (speedup 1.0000x reference)
import functools

import jax
import jax.numpy as jnp
from jax import lax
from jax.experimental import pallas as pl
from jax.experimental.pallas import tpu as pltpu
from jax.experimental.pallas import tpu_sc as plsc

VOCAB = 1000000
EMBED = 64
BATCH = 4096
SEQ = 200

NC = 2
NS = 16
NW = NC * NS

TOTAL = BATCH * SEQ
PER_W = TOTAL // NW
CHUNK = 800
STEPS = PER_W // CHUNK


def _make_kernel():
  mesh = plsc.VectorSubcoreMesh(
      core_axis_name="c", subcore_axis_name="s",
      num_cores=NC, num_subcores=NS)

  @functools.partial(
      pl.kernel,
      out_type=jax.ShapeDtypeStruct((TOTAL, EMBED), jnp.float32),
      mesh=mesh,
      scratch_types=[
          pltpu.VMEM((PER_W,), jnp.int32),
          pltpu.VMEM((2, CHUNK, EMBED), jnp.float32),
          pltpu.SemaphoreType.DMA((2,)),
          pltpu.SemaphoreType.DMA((2,)),
      ],
      compiler_params=pltpu.CompilerParams(use_tc_tiling_on_sc=False),
  )
  def emb_kernel(idx_hbm, table_hbm, out_hbm, idx_v, rows_v, gsem, osem):
    wid = lax.axis_index("s") * NC + lax.axis_index("c")
    base = wid * PER_W

    def idx_slice(i):
      return idx_v.at[pl.ds(i * CHUNK, CHUNK)]

    def out_slice(i):
      return out_hbm.at[pl.ds(base + i * CHUNK, CHUNK)]

    K = 4
    SUB = CHUNK // K

    def start_gather(i, b):
      for k in range(K):
        pltpu.async_copy(
            table_hbm.at[idx_v.at[pl.ds(i * CHUNK + k * SUB, SUB)]],
            rows_v.at[b, pl.ds(k * SUB, SUB)], gsem.at[b])

    def wait_gather(i, b):
      for k in range(K):
        pltpu.make_async_copy(
            table_hbm.at[idx_v.at[pl.ds(i * CHUNK + k * SUB, SUB)]],
            rows_v.at[b, pl.ds(k * SUB, SUB)], gsem.at[b]).wait()

    def start_store(i, b):
      pltpu.async_copy(rows_v.at[b], out_slice(i), osem.at[b])

    def wait_store(i, b):
      pltpu.make_async_copy(rows_v.at[b], out_slice(i), osem.at[b]).wait()

    pltpu.sync_copy(idx_hbm.at[pl.ds(base, PER_W)], idx_v)

    start_gather(0, 0)
    wait_gather(0, 0)
    start_store(0, 0)
    start_gather(1, 1)

    def step(i, _):
      b = lax.rem(i, 2)
      pb = 1 - b
      wait_gather(i - 1, pb)
      start_store(i - 1, pb)
      wait_store(i - 2, b)
      start_gather(i, b)
      return ()

    lax.fori_loop(2, STEPS, step, ())

    last = STEPS - 1
    lb = last % 2
    wait_gather(last, lb)
    start_store(last, lb)
    wait_store(last - 1, 1 - lb)
    wait_store(last, lb)

  return emb_kernel


_emb = _make_kernel()


@jax.jit
def kernel(inputs, table):
  flat_idx = inputs.reshape(TOTAL)
  out = _emb(flat_idx, table)
  return out.reshape(BATCH, SEQ, EMBED)


# speedup vs baseline: 1.0014x; 1.0014x over previous
import functools

import jax
import jax.numpy as jnp
from jax import lax
from jax.experimental import pallas as pl
from jax.experimental.pallas import tpu as pltpu
from jax.experimental.pallas import tpu_sc as plsc

VOCAB = 1000000
EMBED = 64
BATCH = 4096
SEQ = 200

NC = 2
NS = 16
NW = NC * NS

TOTAL = BATCH * SEQ
PER_W = TOTAL // NW
CHUNK = 800
STEPS = PER_W // CHUNK


def _make_kernel():
  mesh = plsc.VectorSubcoreMesh(
      core_axis_name="c", subcore_axis_name="s",
      num_cores=NC, num_subcores=NS)

  @functools.partial(
      pl.kernel,
      out_type=jax.ShapeDtypeStruct((TOTAL, EMBED), jnp.float32),
      mesh=mesh,
      scratch_types=[
          pltpu.VMEM((PER_W,), jnp.int32),
          pltpu.VMEM((2, CHUNK, EMBED), jnp.float32),
          pltpu.SemaphoreType.DMA((2,)),
          pltpu.SemaphoreType.DMA((2,)),
      ],
      compiler_params=pltpu.CompilerParams(use_tc_tiling_on_sc=False),
  )
  def emb_kernel(idx_hbm, table_hbm, out_hbm, idx_v, rows_v, gsem, osem):
    wid = lax.axis_index("s") * NC + lax.axis_index("c")
    base = wid * PER_W

    def idx_slice(i):
      return idx_v.at[pl.ds(i * CHUNK, CHUNK)]

    def out_slice(i):
      return out_hbm.at[pl.ds(base + i * CHUNK, CHUNK)]

    K = 5
    SUB = CHUNK // K

    def start_gather(i, b):
      for k in range(K):
        pltpu.async_copy(
            table_hbm.at[idx_v.at[pl.ds(i * CHUNK + k * SUB, SUB)]],
            rows_v.at[b, pl.ds(k * SUB, SUB)], gsem.at[b])

    def wait_gather(i, b):
      for k in range(K):
        pltpu.make_async_copy(
            table_hbm.at[idx_v.at[pl.ds(i * CHUNK + k * SUB, SUB)]],
            rows_v.at[b, pl.ds(k * SUB, SUB)], gsem.at[b]).wait()

    def start_store(i, b):
      pltpu.async_copy(rows_v.at[b], out_slice(i), osem.at[b])

    def wait_store(i, b):
      pltpu.make_async_copy(rows_v.at[b], out_slice(i), osem.at[b]).wait()

    pltpu.sync_copy(idx_hbm.at[pl.ds(base, PER_W)], idx_v)

    start_gather(0, 0)
    wait_gather(0, 0)
    start_store(0, 0)
    start_gather(1, 1)

    def step(i, _):
      b = lax.rem(i, 2)
      pb = 1 - b
      wait_gather(i - 1, pb)
      start_store(i - 1, pb)
      wait_store(i - 2, b)
      start_gather(i, b)
      return ()

    lax.fori_loop(2, STEPS, step, ())

    last = STEPS - 1
    lb = last % 2
    wait_gather(last, lb)
    start_store(last, lb)
    wait_store(last - 1, 1 - lb)
    wait_store(last, lb)

  return emb_kernel


_emb = _make_kernel()


@jax.jit
def kernel(inputs, table):
  flat_idx = inputs.reshape(TOTAL)
  out = _emb(flat_idx, table)
  return out.reshape(BATCH, SEQ, EMBED)


# speedup vs baseline: 1.0028x; 1.0014x over previous
import functools

import jax
import jax.numpy as jnp
from jax import lax
from jax.experimental import pallas as pl
from jax.experimental.pallas import tpu as pltpu
from jax.experimental.pallas import tpu_sc as plsc

VOCAB = 1000000
EMBED = 64
BATCH = 4096
SEQ = 200

NC = 2
NS = 16
NW = NC * NS

TOTAL = BATCH * SEQ
PER_W = TOTAL // NW
CHUNK = 800
STEPS = PER_W // CHUNK


def _make_kernel():
  mesh = plsc.VectorSubcoreMesh(
      core_axis_name="c", subcore_axis_name="s",
      num_cores=NC, num_subcores=NS)

  @functools.partial(
      pl.kernel,
      out_type=jax.ShapeDtypeStruct((TOTAL, EMBED), jnp.float32),
      mesh=mesh,
      scratch_types=[
          pltpu.VMEM((PER_W,), jnp.int32),
          pltpu.VMEM((2, CHUNK, EMBED), jnp.float32),
          pltpu.SemaphoreType.DMA((2,)),
          pltpu.SemaphoreType.DMA((2,)),
      ],
      compiler_params=pltpu.CompilerParams(use_tc_tiling_on_sc=False),
  )
  def emb_kernel(idx_hbm, table_hbm, out_hbm, idx_v, rows_v, gsem, osem):
    wid = lax.axis_index("s") * NC + lax.axis_index("c")
    base = wid * PER_W

    def idx_slice(i):
      return idx_v.at[pl.ds(i * CHUNK, CHUNK)]

    def out_slice(i):
      return out_hbm.at[pl.ds(base + i * CHUNK, CHUNK)]

    K = 10
    SUB = CHUNK // K

    def start_gather(i, b):
      for k in range(K):
        pltpu.async_copy(
            table_hbm.at[idx_v.at[pl.ds(i * CHUNK + k * SUB, SUB)]],
            rows_v.at[b, pl.ds(k * SUB, SUB)], gsem.at[b])

    def wait_gather(i, b):
      for k in range(K):
        pltpu.make_async_copy(
            table_hbm.at[idx_v.at[pl.ds(i * CHUNK + k * SUB, SUB)]],
            rows_v.at[b, pl.ds(k * SUB, SUB)], gsem.at[b]).wait()

    def start_store(i, b):
      pltpu.async_copy(rows_v.at[b], out_slice(i), osem.at[b])

    def wait_store(i, b):
      pltpu.make_async_copy(rows_v.at[b], out_slice(i), osem.at[b]).wait()

    pltpu.sync_copy(idx_hbm.at[pl.ds(base, PER_W)], idx_v)

    start_gather(0, 0)
    wait_gather(0, 0)
    start_store(0, 0)
    start_gather(1, 1)

    def step(i, _):
      b = lax.rem(i, 2)
      pb = 1 - b
      wait_gather(i - 1, pb)
      start_store(i - 1, pb)
      wait_store(i - 2, b)
      start_gather(i, b)
      return ()

    lax.fori_loop(2, STEPS, step, ())

    last = STEPS - 1
    lb = last % 2
    wait_gather(last, lb)
    start_store(last, lb)
    wait_store(last - 1, 1 - lb)
    wait_store(last, lb)

  return emb_kernel


_emb = _make_kernel()


@jax.jit
def kernel(inputs, table):
  flat_idx = inputs.reshape(TOTAL)
  out = _emb(flat_idx, table)
  return out.reshape(BATCH, SEQ, EMBED)
